# Initial kernel scaffold; baseline (speedup 1.0000x reference)
import jax
import jax.numpy as jnp
from jax.experimental import pallas as pl


def kernel(user_indices, item_indices, user_tables, item_tables, user_W1, user_b1, user_W2, user_b2, item_W1, item_b1, item_W2, item_b2):
    raise NotImplementedError("write your pallas kernel here")


# speedup vs baseline: 7.1390x; 7.1390x over previous
import functools

import jax
import jax.numpy as jnp
from jax import lax
from jax.experimental import pallas as pl
from jax.experimental.pallas import tpu as pltpu
from jax.experimental.pallas import tpu_sc as plsc

N_FIELD = 13
VOCAB = 100000
EMB = 16
B = 16384
U_IN = N_FIELD * EMB
H1, H2 = 64, 32

NC = 2
NS = 16
NW = NC * NS
BPW = B // NW
CH = 128
NCH = BPW // CH


def _sc_gather_body(utab, itab, uidx, iidx, uout, iout, idx_v, rows_v, sem):
    c = lax.axis_index("c")
    s = lax.axis_index("s")
    wid = s * NC + c
    base = wid * BPW

    def one_field(f, tab, idx_hbm, out_hbm):
        pltpu.sync_copy(idx_hbm.at[f, wid], idx_v)
        copies = []
        for j in range(NCH):
            copies.append(
                pltpu.async_copy(tab.at[idx_v.at[j]],
                                 rows_v.at[pl.ds(j * CH, CH)], sem))
        for cp in copies:
            cp.wait()
        pltpu.sync_copy(rows_v, out_hbm.at[f, pl.ds(base, BPW)])

    def uloop(f, carry):
        one_field(f, utab, uidx, uout)
        return carry

    def iloop(f, carry):
        one_field(f, itab, iidx, iout)
        return carry

    lax.fori_loop(0, N_FIELD, uloop, 0, unroll=False)
    lax.fori_loop(0, N_FIELD, iloop, 0, unroll=False)


def _sc_gather(utab, itab, uidx, iidx):
    mesh = plsc.VectorSubcoreMesh(core_axis_name="c", subcore_axis_name="s")
    f = pl.kernel(
        _sc_gather_body,
        out_type=[
            jax.ShapeDtypeStruct((N_FIELD, B, EMB), jnp.float32),
            jax.ShapeDtypeStruct((N_FIELD, B, EMB), jnp.float32),
        ],
        mesh=mesh,
        scratch_types=[
            pltpu.VMEM((NCH, CH), jnp.int32),
            pltpu.VMEM((BPW, EMB), jnp.float32),
            pltpu.SemaphoreType.DMA,
        ],
        compiler_params=pltpu.CompilerParams(use_tc_tiling_on_sc=False),
    )
    return f(utab, itab, uidx, iidx)


BB = 512


def _tc_dnn_body(uemb, iemb, uW1, ub1, uW2, ub2, iW1, ib1, iW2, ib2,
                 out, acc):
    i = pl.program_id(0)

    @pl.when(i == 0)
    def _init():
        acc[0] = 0.0
        acc[1] = 0.0
        acc[2] = 0.0

    def tower(emb, W1, b1, W2, b2):
        x = jnp.dot(emb[0], W1[0:EMB, :], preferred_element_type=jnp.float32)
        for f in range(1, N_FIELD):
            x = x + jnp.dot(emb[f], W1[f * EMB:(f + 1) * EMB, :],
                            preferred_element_type=jnp.float32)
        x = jnp.maximum(x + b1[0:1, :], 0.0)
        h = jnp.dot(x, W2[...], preferred_element_type=jnp.float32)
        return jnp.maximum(h + b2[0:1, :], 0.0)

    u = tower(uemb, uW1, ub1, uW2, ub2)
    v = tower(iemb, iW1, ib1, iW2, ib2)

    acc[0] += jnp.sum(u * v)
    acc[1] += jnp.sum(u * u)
    acc[2] += jnp.sum(v * v)

    @pl.when(i == pl.num_programs(0) - 1)
    def _fin():
        cos = acc[0] / jnp.sqrt(acc[1] * acc[2])
        out[0, 0] = 1.0 / (1.0 + jnp.exp(-cos))


def _tc_dnn(uemb, iemb, uW1, ub1, uW2, ub2, iW1, ib1, iW2, ib2):
    nblk = B // BB
    emb_spec = pl.BlockSpec((N_FIELD, BB, EMB), lambda i: (0, i, 0))
    full = lambda shape: pl.BlockSpec(shape, lambda i: (0, 0))
    return pl.pallas_call(
        _tc_dnn_body,
        grid=(nblk,),
        in_specs=[
            emb_spec, emb_spec,
            full((U_IN, H1)), full((1, H1)), full((H1, H2)), full((1, H2)),
            full((U_IN, H1)), full((1, H1)), full((H1, H2)), full((1, H2)),
        ],
        out_specs=pl.BlockSpec((1, 1), lambda i: (0, 0),
                               memory_space=pltpu.SMEM),
        out_shape=jax.ShapeDtypeStruct((1, 1), jnp.float32),
        scratch_shapes=[pltpu.SMEM((3,), jnp.float32)],
        compiler_params=pltpu.CompilerParams(
            dimension_semantics=("arbitrary",)),
    )(uemb, iemb, uW1, ub1, uW2, ub2, iW1, ib1, iW2, ib2)


def kernel(user_indices, item_indices, user_tables, item_tables,
           user_W1, user_b1, user_W2, user_b2,
           item_W1, item_b1, item_W2, item_b2):
    offs = (jnp.arange(N_FIELD, dtype=jnp.int32) * VOCAB)[:, None]
    uidx = (user_indices + offs).reshape(N_FIELD, NW, NCH, CH)
    iidx = (item_indices + offs).reshape(N_FIELD, NW, NCH, CH)
    utab = user_tables.reshape(N_FIELD * VOCAB, EMB)
    itab = item_tables.reshape(N_FIELD * VOCAB, EMB)

    uemb, iemb = _sc_gather(utab, itab, uidx, iidx)

    return _tc_dnn(uemb, iemb,
                   user_W1, user_b1.reshape(1, H1),
                   user_W2, user_b2.reshape(1, H2),
                   item_W1, item_b1.reshape(1, H1),
                   item_W2, item_b2.reshape(1, H2))


# speedup vs baseline: 45.3009x; 6.3455x over previous
import functools

import jax
import jax.numpy as jnp
from jax import lax
from jax.experimental import pallas as pl
from jax.experimental.pallas import tpu as pltpu
from jax.experimental.pallas import tpu_sc as plsc

N_FIELD = 13
VOCAB = 100000
EMB = 16
B = 16384
H1, H2 = 64, 32

NC = 2
NS = 16
IC = 8192
NIC = B // IC
GU = 32


def _sc_gather_body(utab, itab, uidx, iidx, uout, iout,
                    row_v, idx_v, out_v):
    c = lax.axis_index("c")
    s = lax.axis_index("s")

    def tower_work(tab, idx_hbm, out_hbm):
        def field_loop(f, carry):
            pltpu.sync_copy(tab.at[f, s], row_v)

            def chunk(ci, carry2):
                pltpu.sync_copy(idx_hbm.at[f, pl.ds(ci * IC, IC)], idx_v)

                def gloop(g, carry3):
                    base = g * (GU * 16)
                    for u in range(GU):
                        iv = idx_v[pl.ds(base + u * 16, 16)]
                        vals = plsc.load_gather(row_v, [iv])
                        out_v[pl.ds(base + u * 16, 16)] = vals
                    return carry3

                lax.fori_loop(0, IC // (GU * 16), gloop, 0, unroll=False)
                pltpu.sync_copy(out_v, out_hbm.at[f, s, pl.ds(ci * IC, IC)])
                return carry2

            lax.fori_loop(0, NIC, chunk, 0, unroll=False)
            return carry

        lax.fori_loop(0, N_FIELD, field_loop, 0, unroll=False)

    def user_branch():
        tower_work(utab, uidx, uout)

    def item_branch():
        tower_work(itab, iidx, iout)

    lax.cond(c == 0, user_branch, item_branch)


def _sc_gather(utabT, itabT, uidx, iidx):
    mesh = plsc.VectorSubcoreMesh(core_axis_name="c", subcore_axis_name="s")
    f = pl.kernel(
        _sc_gather_body,
        out_type=[
            jax.ShapeDtypeStruct((N_FIELD, EMB, B), jnp.float32),
            jax.ShapeDtypeStruct((N_FIELD, EMB, B), jnp.float32),
        ],
        mesh=mesh,
        scratch_types=[
            pltpu.VMEM((VOCAB,), jnp.float32),
            pltpu.VMEM((IC,), jnp.int32),
            pltpu.VMEM((IC,), jnp.float32),
        ],
        compiler_params=pltpu.CompilerParams(use_tc_tiling_on_sc=True,
                                             needs_layout_passes=False),
    )
    return f(utabT, itabT, uidx, iidx)


BB = 1024


def _tc_dnn_body(uemb, iemb, uW1T, ub1, uW2T, ub2, iW1T, ib1, iW2T, ib2,
                 out, acc):
    i = pl.program_id(0)

    @pl.when(i == 0)
    def _init():
        acc[0] = 0.0
        acc[1] = 0.0
        acc[2] = 0.0

    def tower(emb, W1T, b1, W2T, b2):
        e = emb[...].reshape(N_FIELD * EMB, BB)
        x = jnp.dot(W1T[...], e, preferred_element_type=jnp.float32)
        x = jnp.maximum(x + b1[...], 0.0)
        h = jnp.dot(W2T[...], x, preferred_element_type=jnp.float32)
        return jnp.maximum(h + b2[...], 0.0)

    u = tower(uemb, uW1T, ub1, uW2T, ub2)
    v = tower(iemb, iW1T, ib1, iW2T, ib2)

    acc[0] += jnp.sum(u * v)
    acc[1] += jnp.sum(u * u)
    acc[2] += jnp.sum(v * v)

    @pl.when(i == pl.num_programs(0) - 1)
    def _fin():
        cos = acc[0] / jnp.sqrt(acc[1] * acc[2])
        out[0, 0] = 1.0 / (1.0 + jnp.exp(-cos))


def _tc_dnn(uemb, iemb, uW1T, ub1, uW2T, ub2, iW1T, ib1, iW2T, ib2):
    nblk = B // BB
    emb_spec = pl.BlockSpec((N_FIELD, EMB, BB), lambda i: (0, 0, i))
    full = lambda shape: pl.BlockSpec(shape, lambda i: (0,) * len(shape))
    return pl.pallas_call(
        _tc_dnn_body,
        grid=(nblk,),
        in_specs=[
            emb_spec, emb_spec,
            full((H1, N_FIELD * EMB)), full((H1, 1)),
            full((H2, H1)), full((H2, 1)),
            full((H1, N_FIELD * EMB)), full((H1, 1)),
            full((H2, H1)), full((H2, 1)),
        ],
        out_specs=pl.BlockSpec((1, 1), lambda i: (0, 0),
                               memory_space=pltpu.SMEM),
        out_shape=jax.ShapeDtypeStruct((1, 1), jnp.float32),
        scratch_shapes=[pltpu.SMEM((3,), jnp.float32)],
        compiler_params=pltpu.CompilerParams(
            dimension_semantics=("arbitrary",)),
    )(uemb, iemb, uW1T, ub1, uW2T, ub2, iW1T, ib1, iW2T, ib2)


def kernel(user_indices, item_indices, user_tables, item_tables,
           user_W1, user_b1, user_W2, user_b2,
           item_W1, item_b1, item_W2, item_b2):
    utabT = jnp.transpose(user_tables, (0, 2, 1))
    itabT = jnp.transpose(item_tables, (0, 2, 1))

    uembT, iembT = _sc_gather(utabT, itabT, user_indices, item_indices)

    return _tc_dnn(uembT, iembT,
                   user_W1.T, user_b1.reshape(H1, 1),
                   user_W2.T, user_b2.reshape(H2, 1),
                   item_W1.T, item_b1.reshape(H1, 1),
                   item_W2.T, item_b2.reshape(H2, 1))
